# baseline (speedup 1.0000x reference)
import functools

import jax
import jax.numpy as jnp
from jax import lax
from jax.experimental import pallas as pl
from jax.experimental.pallas import tpu as pltpu
from jax.experimental.pallas import tpu_sc as plsc

B = 16384
N = 200
D = 128
V = 400
NP = 208
NV = NP // 16
L = 16

_info = plsc.get_sparse_core_info()
NC, NS = _info.num_cores, _info.num_subcores
NW = NC * NS
B_PER_W = B // NW
CB = 16
N_CHUNKS = B_PER_W // CB

_mesh = plsc.VectorSubcoreMesh(core_axis_name="c", subcore_axis_name="s")


@functools.partial(
    pl.kernel,
    mesh=_mesh,
    out_type=jax.ShapeDtypeStruct((B * D * N,), jnp.float32),
    scratch_types=[
        pltpu.VMEM((V * D,), jnp.float32),
        pltpu.VMEM((CB * N * 2,), jnp.int32),
        pltpu.VMEM((CB * NP,), jnp.int32),
        pltpu.VMEM((D * N + L,), jnp.float32),
    ],
    compiler_params=pltpu.CompilerParams(needs_layout_passes=False),
)
def _sc_embed(xy_hbm, emb_hbm, out_hbm, table_v, xy_v, idx_v, ob):
    wid = lax.axis_index("s") * NC + lax.axis_index("c")
    b0w = wid * B_PER_W
    pltpu.sync_copy(emb_hbm, table_v)
    iota = jnp.arange(L, dtype=jnp.int32)

    def chunk_body(ci, _):
        b0 = b0w + ci * CB
        pltpu.sync_copy(xy_hbm.at[pl.ds(b0 * N * 2, CB * N * 2)], xy_v)

        def row_body(r, _):
            def vec_body(k, _):
                n = jnp.minimum(k * L + iota, N - 1)
                src = (r * N + n) * 2
                xv = plsc.load_gather(xy_v, [src])
                yv = plsc.load_gather(xy_v, [src + 1])
                idx_v[pl.ds(r * NP + k * L, L)] = (xv * 20 + yv) * D
                return 0

            return lax.fori_loop(0, NV, vec_body, 0, unroll=True)

        lax.fori_loop(0, CB, row_body, 0)

        def b_body(bl, _):
            base = bl * NP
            vecs = tuple(idx_v[pl.ds(base + k * L, L)] for k in range(NV))

            def d_body(d, vs):
                for k in range(NV):
                    ob[pl.ds(d * N + k * L, L)] = plsc.load_gather(
                        table_v, [vs[k]])
                return tuple(v + 1 for v in vs)

            lax.fori_loop(0, D, d_body, vecs)
            pltpu.sync_copy(ob.at[pl.ds(0, D * N)],
                            out_hbm.at[pl.ds((b0 + bl) * D * N, D * N)])
            return 0

        lax.fori_loop(0, CB, b_body, 0)
        return 0

    lax.fori_loop(0, N_CHUNKS, chunk_body, 0)


def kernel(xy, embedding):
    out = _sc_embed(xy.reshape(-1), embedding.reshape(-1))
    return out.reshape(B, D, N)


# speedup vs baseline: 1.2028x; 1.2028x over previous
import functools

import jax
import jax.numpy as jnp
from jax import lax
from jax.experimental import pallas as pl
from jax.experimental.pallas import tpu as pltpu
from jax.experimental.pallas import tpu_sc as plsc

B = 16384
N = 200
D = 128
V = 400
NP = 208
NV = NP // 16
NF = 12
L = 16

_info = plsc.get_sparse_core_info()
NC, NS = _info.num_cores, _info.num_subcores
NW = NC * NS
B_PER_W = B // NW
CB = 16
N_CHUNKS = B_PER_W // CB

_mesh = plsc.VectorSubcoreMesh(core_axis_name="c", subcore_axis_name="s")


@functools.partial(
    pl.kernel,
    mesh=_mesh,
    out_type=jax.ShapeDtypeStruct((B, D, N), jnp.float32),
    scratch_types=[
        pltpu.VMEM((V * D,), jnp.float32),
        pltpu.VMEM((CB * N * 2,), jnp.int32),
        pltpu.VMEM((CB * NP,), jnp.int32),
        pltpu.VMEM((1, D, N), jnp.float32),
    ],
    compiler_params=pltpu.CompilerParams(needs_layout_passes=False),
)
def _sc_embed(xy_hbm, emb_hbm, out_hbm, table_v, xy_v, idx_v, ob):
    wid = lax.axis_index("s") * NC + lax.axis_index("c")
    b0w = wid * B_PER_W
    pltpu.sync_copy(emb_hbm, table_v)
    iota = jnp.arange(L, dtype=jnp.int32)
    lo8 = iota & 7
    hi8 = iota >> 3
    zero16 = iota * 0
    ntail = 192 + lo8

    def chunk_body(ci, _):
        b0 = b0w + ci * CB
        pltpu.sync_copy(xy_hbm.at[pl.ds(b0 * N * 2, CB * N * 2)], xy_v)

        def row_body(r, _):
            for k in range(NV):
                n = 192 + lo8 if k == NV - 1 else k * L + iota
                src = (r * N + n) * 2
                xv = plsc.load_gather(xy_v, [src])
                yv = plsc.load_gather(xy_v, [src + 1])
                idx_v[pl.ds(r * NP + k * L, L)] = (xv * 20 + yv) * D
            return 0

        lax.fori_loop(0, CB, row_body, 0)

        def b_body(bl, _):
            base = bl * NP
            vecs = tuple(idx_v[pl.ds(base + k * L, L)] for k in range(NV))

            def dpair_body(j, vs):
                d0 = j * 2
                for k in range(NF):
                    ob[0, d0, pl.ds(k * L, L)] = plsc.load_gather(
                        table_v, [vs[k]])
                for k in range(NF):
                    ob[0, d0 + 1, pl.ds(k * L, L)] = plsc.load_gather(
                        table_v, [vs[k] + 1])
                tvals = plsc.load_gather(table_v, [vs[NF] + hi8])
                plsc.store_scatter(ob, [zero16, d0 + hi8, ntail], tvals)
                return tuple(v + 2 for v in vs)

            lax.fori_loop(0, D // 2, dpair_body, vecs)
            pltpu.sync_copy(ob, out_hbm.at[pl.ds(b0 + bl, 1)])
            return 0

        lax.fori_loop(0, CB, b_body, 0)
        return 0

    lax.fori_loop(0, N_CHUNKS, chunk_body, 0)


def kernel(xy, embedding):
    return _sc_embed(xy.reshape(-1), embedding.reshape(-1))


# speedup vs baseline: 6.3216x; 5.2558x over previous
import functools

import jax
import jax.numpy as jnp
from jax import lax
from jax.experimental import pallas as pl
from jax.experimental.pallas import tpu as pltpu
from jax.experimental.pallas import tpu_sc as plsc

B = 16384
N = 200
D = 128
V = 400
L = 16
K = 128

_info = plsc.get_sparse_core_info()
NC, NS = _info.num_cores, _info.num_subcores
NW = NC * NS
B_PER_W = B // NW
NG = 4
GB = 128
ROWS_G = GB * N
STEPS = ROWS_G // K
WIN = 16

_mesh = plsc.VectorSubcoreMesh(core_axis_name="c", subcore_axis_name="s")


@functools.partial(
    pl.kernel,
    mesh=_mesh,
    out_type=jax.ShapeDtypeStruct((B * N, D), jnp.float32),
    scratch_types=[
        pltpu.VMEM((N * 2 * GB,), jnp.int32),
        pltpu.VMEM((GB * N,), jnp.int32),
        pltpu.VMEM((K, D), jnp.float32),
        pltpu.VMEM((K, D), jnp.float32),
        pltpu.SemaphoreType.DMA,
        pltpu.SemaphoreType.DMA,
        pltpu.SemaphoreType.DMA,
        pltpu.SemaphoreType.DMA,
    ],
    compiler_params=pltpu.CompilerParams(needs_layout_passes=False),
)
def _sc_embed(xy_hbm, emb_hbm, out_hbm, xy_s, idxbuf, rb0, rb1,
              sem_x, sem_g, sem_s0, sem_s1):
    wid = lax.axis_index("s") * NC + lax.axis_index("c")
    bt0 = wid * NG
    iota = jnp.arange(L, dtype=jnp.int32)
    lane_base = iota * N

    def group_body(g, _):
        bt = bt0 + g

        def stage(n, _):
            pltpu.async_copy(
                xy_hbm.at[pl.ds(n * (128 * 2 * GB) + bt * (2 * GB), 2 * GB)],
                xy_s.at[pl.ds(n * (2 * GB), 2 * GB)], sem_x)

            @pl.when(n >= WIN)
            def _wait():
                pltpu.make_async_copy(xy_hbm.at[pl.ds(0, 2 * GB)],
                                      xy_s.at[pl.ds(0, 2 * GB)], sem_x).wait()
            return 0

        lax.fori_loop(0, N, stage, 0)

        def stage_drain(i, _):
            pltpu.make_async_copy(xy_hbm.at[pl.ds(0, 2 * GB)],
                                  xy_s.at[pl.ds(0, 2 * GB)], sem_x).wait()
            return 0

        lax.fori_loop(0, WIN, stage_drain, 0)

        def ib(n, _):
            base = n * (2 * GB)
            for h in range(GB // L):
                x16 = xy_s[pl.ds(base + h * L, L)]
                y16 = xy_s[pl.ds(base + GB + h * L, L)]
                plsc.store_scatter(idxbuf, [lane_base + (h * L * N + n)],
                                   x16 * 20 + y16)
            return 0

        lax.fori_loop(0, N, ib, 0)

        r0 = bt * ROWS_G

        def rounds(s2, _):
            for rb, sem_s, t in ((rb0, sem_s0, 0), (rb1, sem_s1, 1)):
                s = s2 * 2 + t

                @pl.when(s2 > 0)
                def _wait_sc():
                    pltpu.make_async_copy(rb, out_hbm.at[pl.ds(0, K)],
                                          sem_s).wait()

                pltpu.async_copy(
                    emb_hbm.at[idxbuf.at[pl.ds(s * K, K)]], rb, sem_g).wait()
                pltpu.async_copy(rb, out_hbm.at[pl.ds(r0 + s * K, K)], sem_s)
            return 0

        lax.fori_loop(0, STEPS // 2, rounds, 0)
        pltpu.make_async_copy(rb0, out_hbm.at[pl.ds(0, K)], sem_s0).wait()
        pltpu.make_async_copy(rb1, out_hbm.at[pl.ds(0, K)], sem_s1).wait()
        return 0

    lax.fori_loop(0, NG, group_body, 0)


def kernel(xy, embedding):
    xyf = (xy.transpose(1, 0, 2)
             .reshape(N, 128, 128, 2)
             .transpose(0, 1, 3, 2)
             .reshape(-1))
    out = _sc_embed(xyf, embedding)
    return out.reshape(B, N, D).transpose(0, 2, 1)
